# baseline (speedup 1.0000x reference)
import functools

import jax
import jax.numpy as jnp
from jax import lax
from jax.experimental import pallas as pl
from jax.experimental.pallas import tpu as pltpu
from jax.experimental.pallas import tpu_sc as plsc

N = 10000
E = 320000
D = 128
H = 128
C = 10

NC = 2
NS = 16
NW = NC * NS
EW = E // NW
K = 40
NCHUNK = EW // K
J = 10
NBLK = NCHUNK // J
NPAD = 10112
RPT = NPAD // NS


def _tc_prep(x_ref, we_ref, be_ref, tsrc_ref, tdst_ref):
    x = x_ref[...]
    a = jnp.dot(x, we_ref[:D, :], preferred_element_type=jnp.float32)
    b = jnp.dot(x, we_ref[D:, :], preferred_element_type=jnp.float32) + be_ref[...]
    tsrc_ref[:, :D] = x
    tsrc_ref[:, D:] = a
    tdst_ref[...] = b


def _sc_edge_body(tsrc_hbm, tdst_hbm, srcb_hbm, dstb_hbm, zero_hbm, out_hbm,
                  sblks, dblks, sbufs, dbufs, acc, isems, gsems, ssems):
    c = lax.axis_index("c")
    s = lax.axis_index("s")
    w = s * NC + c
    pltpu.sync_copy(zero_hbm.at[pl.ds(s * RPT, RPT)], acc.at[pl.ds(s * RPT, RPT)])
    plsc.subcore_barrier()

    def fire_idxblk(m, slot):
        row = w * NBLK + m
        pltpu.async_copy(srcb_hbm.at[row], sblks[slot], isems[slot])
        pltpu.async_copy(dstb_hbm.at[row], dblks[slot], isems[slot])

    def wait_idxblk(slot):
        pltpu.make_async_copy(srcb_hbm.at[0], sblks[slot], isems[slot]).wait()
        pltpu.make_async_copy(dstb_hbm.at[0], dblks[slot], isems[slot]).wait()

    def fire_gather(b, sref, dref):
        pltpu.async_copy(tsrc_hbm.at[sref], sbufs[b], gsems[b])
        pltpu.async_copy(tdst_hbm.at[dref], dbufs[b], gsems[b])

    def wait_gather(b):
        pltpu.make_async_copy(tsrc_hbm.at[sblks[0].at[0]], sbufs[b], gsems[b]).wait()
        pltpu.make_async_copy(tdst_hbm.at[dblks[0].at[0]], dbufs[b], gsems[b]).wait()

    def fire_scatter(b, dref):
        pltpu.async_copy(dbufs[b], acc.at[dref], ssems[b], add=True)

    def wait_scatter(b):
        pltpu.make_async_copy(dbufs[b], acc.at[dblks[0].at[0]], ssems[b]).wait()

    def compute(b):
        srows, drows = sbufs[b], dbufs[b]

        @plsc.parallel_loop(0, K, 1, unroll=2)
        def _(k):
            for j in range(H // 16):
                xv = srows[k, pl.ds(j * 16, 16)]
                av = srows[k, pl.ds(D + j * 16, 16)]
                bv = drows[k, pl.ds(j * 16, 16)]
                drows[k, pl.ds(j * 16, 16)] = xv * jnp.maximum(av + bv, 0.0)

    def blk(m, islot, first=False, guard=False):
        nslot = 1 - islot
        sb_c, db_c = sblks[islot], dblks[islot]
        sb_n, db_n = sblks[nslot], dblks[nslot]

        def next_blk_idx():
            fire_idxblk(m + 1, nslot)

        def cross_gather():
            wait_idxblk(nslot)
            fire_gather(0, sb_n.at[0], db_n.at[0])

        def pair(tp, carry):
            t0 = 2 * tp
            if first:
                pl.when(tp >= 1)(lambda: wait_scatter(1))
            else:
                wait_scatter(1)

            @pl.when(tp == 0)
            def _():
                if guard:
                    pl.when(m + 1 < NBLK)(next_blk_idx)
                else:
                    next_blk_idx()

            fire_gather(1, sb_c.at[t0 + 1], db_c.at[t0 + 1])
            wait_gather(0)
            compute(0)
            fire_scatter(0, db_c.at[t0])

            wait_scatter(0)

            @pl.when(tp == J // 2 - 1)
            def _():
                if guard:
                    pl.when(m + 1 < NBLK)(cross_gather)
                else:
                    cross_gather()

            @pl.when(tp < J // 2 - 1)
            def _():
                fire_gather(0, sb_c.at[t0 + 2], db_c.at[t0 + 2])

            wait_gather(1)
            compute(1)
            fire_scatter(1, db_c.at[t0 + 1])
            return carry

        lax.fori_loop(0, J // 2, pair, 0)

    fire_idxblk(0, 0)
    wait_idxblk(0)
    fire_gather(0, sblks[0].at[0], dblks[0].at[0])

    blk(0, 0, first=True)

    def outer(mm, carry):
        blk(1 + 2 * mm, 1)
        blk(2 + 2 * mm, 0, guard=True)
        return carry

    lax.fori_loop(0, (NBLK - 1) // 2, outer, 0)
    wait_scatter(1)

    plsc.subcore_barrier()
    pltpu.sync_copy(acc.at[pl.ds(s * RPT, RPT)], out_hbm.at[c, pl.ds(s * RPT, RPT)])


def _sc_edge_entry(tsrc_hbm, tdst_hbm, srcb_hbm, dstb_hbm, zero_hbm, out_hbm,
                   sb0, sb1, db0, db1, s0, s1, d0, d1, acc,
                   i0, i1, g0, g1, x0, x1):
    _sc_edge_body(tsrc_hbm, tdst_hbm, srcb_hbm, dstb_hbm, zero_hbm, out_hbm,
                  (sb0, sb1), (db0, db1), (s0, s1), (d0, d1), acc,
                  (i0, i1), (g0, g1), (x0, x1))


@functools.cache
def _sc_edge():
    return pl.kernel(
        _sc_edge_entry,
        out_type=jax.ShapeDtypeStruct((NC, NPAD, H), jnp.float32),
        mesh=plsc.VectorSubcoreMesh(core_axis_name="c", subcore_axis_name="s",
                                    num_cores=NC, num_subcores=NS),
        scratch_types=(
            [pltpu.VMEM((J, K), jnp.int32) for _ in range(4)]
            + [pltpu.VMEM((K, 2 * D), jnp.float32) for _ in range(2)]
            + [pltpu.VMEM((K, H), jnp.float32) for _ in range(2)]
            + [pltpu.VMEM_SHARED((NPAD, H), jnp.float32)]
            + [pltpu.SemaphoreType.DMA for _ in range(6)]
        ),
    )


def _tc_finish(p_ref, wn_ref, bn_ref, wg_ref, bg_ref, wf_ref, bf_ref, out_ref):
    hn = p_ref[0, :N, :] + p_ref[1, :N, :]
    h = jnp.maximum(
        jnp.dot(hn, wn_ref[...], preferred_element_type=jnp.float32) + bn_ref[...],
        0.0)
    g = jnp.sum(h * wg_ref[...], axis=1, keepdims=True) + bg_ref[...]
    m = jnp.max(g)
    e = jnp.exp(g - m)
    ssum = jnp.sum(e)
    r = jnp.sum(e * h, axis=0, keepdims=True) / ssum
    out_ref[...] = jnp.dot(r, wf_ref[...], preferred_element_type=jnp.float32) + bf_ref[...]


def kernel(x, edge_index, W_edge, b_edge, W_node, b_node, W_gate, b_gate, W_fc, b_fc):
    tsrc, tdst = pl.pallas_call(
        _tc_prep,
        out_shape=[
            jax.ShapeDtypeStruct((N, 2 * D), jnp.float32),
            jax.ShapeDtypeStruct((N, H), jnp.float32),
        ],
    )(x, W_edge, b_edge.reshape(1, H))
    srcb = edge_index[0].reshape(NW * NBLK, J, K)
    dstb = edge_index[1].reshape(NW * NBLK, J, K)
    p = _sc_edge()(tsrc, tdst, srcb, dstb, jnp.zeros((NPAD, H), jnp.float32))
    out = pl.pallas_call(
        _tc_finish,
        out_shape=jax.ShapeDtypeStruct((1, C), jnp.float32),
    )(p, W_node, b_node.reshape(1, H), W_gate.reshape(1, H),
      b_gate.reshape(1, 1), W_fc, b_fc.reshape(1, C))
    return out
